# Initial kernel scaffold; baseline (speedup 1.0000x reference)
import jax
import jax.numpy as jnp
from jax.experimental import pallas as pl


def kernel(x, edge_index, params):
    raise NotImplementedError("write your pallas kernel here")


# speedup vs baseline: 4.4926x; 4.4926x over previous
import functools

import jax
import jax.numpy as jnp
from jax import lax
from jax.experimental import pallas as pl
from jax.experimental.pallas import tpu as pltpu
from jax.experimental.pallas import tpu_sc as plsc

_N = 10000
_E = 320000
_D = 128
_H = 128
_OUT = 16
_NUM_GIN = 6
_EPS = 1e-5

_NC = 2
_NS = 16
_NW = _NC * _NS
_K = 128
_EPT = _E // _NW
_NCHUNK = -(-_EPT // _K)
_PAD = _NCHUNK * _K - _EPT
_NPAD = 16
_RPT = _N // _NS


def _sc_agg_body(h_hbm, zeros_hbm, src_hbm, dst_hbm, out0_hbm, out1_hbm,
                 src_v, dst_v, rows_v, acc_sh, sem):
  c = lax.axis_index("c")
  s = lax.axis_index("s")
  wid = c * _NS + s

  @pl.when(jnp.logical_and(c == 0, s == 0))
  def _():
    pltpu.sync_copy(h_hbm, acc_sh.at[pl.ds(0, _N)])
    pltpu.sync_copy(zeros_hbm.at[pl.ds(_N, _NPAD)],
                    acc_sh.at[pl.ds(_N, _NPAD)])

  @pl.when(jnp.logical_and(c == 1, s == 0))
  def _():
    pltpu.sync_copy(zeros_hbm, acc_sh)

  pltpu.sync_copy(src_hbm.at[wid], src_v)
  pltpu.sync_copy(dst_hbm.at[wid], dst_v)
  plsc.subcore_barrier()

  def chunk(j, carry):
    pltpu.async_copy(h_hbm.at[src_v.at[j]], rows_v, sem).wait()
    pltpu.sync_copy(rows_v, acc_sh.at[dst_v.at[j]], add=True)
    return carry

  lax.fori_loop(0, _NCHUNK, chunk, 0)
  plsc.subcore_barrier()

  @pl.when(jnp.logical_and(c == 0, s == 0))
  def _():
    pltpu.sync_copy(acc_sh.at[pl.ds(0, _N)], out0_hbm)

  @pl.when(jnp.logical_and(c == 1, s == 0))
  def _():
    pltpu.sync_copy(acc_sh.at[pl.ds(0, _N)], out1_hbm)


@functools.lru_cache(maxsize=None)
def _make_sc_agg():
  return pl.kernel(
      _sc_agg_body,
      out_type=(jax.ShapeDtypeStruct((_N, _D), jnp.float32),
                jax.ShapeDtypeStruct((_N, _D), jnp.float32)),
      mesh=plsc.VectorSubcoreMesh(core_axis_name="c", subcore_axis_name="s",
                                  num_cores=_NC, num_subcores=_NS),
      scratch_types=[
          pltpu.VMEM((_NCHUNK, _K), jnp.int32),
          pltpu.VMEM((_NCHUNK, _K), jnp.int32),
          pltpu.VMEM((_K, _D), jnp.float32),
          pltpu.VMEM_SHARED((_N + _NPAD, _D), jnp.float32),
          pltpu.SemaphoreType.DMA,
      ],
  )


def _sc_agg(h, zeros, src, dst):
  return _make_sc_agg()(h, zeros, src, dst)


def _mlp_bn(z, w1t, bng, bnb, w2t, bg, bb):
  z1 = jnp.dot(z, w1t, preferred_element_type=jnp.float32)
  m = jnp.mean(z1, axis=0, keepdims=True)
  d = z1 - m
  v = jnp.mean(d * d, axis=0, keepdims=True)
  a = jnp.maximum(d * lax.rsqrt(v + _EPS) * bng + bnb, 0.0)
  z2 = jnp.dot(a, w2t, preferred_element_type=jnp.float32)
  m2 = jnp.mean(z2, axis=0, keepdims=True)
  d2 = z2 - m2
  v2 = jnp.mean(d2 * d2, axis=0, keepdims=True)
  return jnp.maximum(d2 * lax.rsqrt(v2 + _EPS) * bg + bb, 0.0)


def _tc_layer_body(p0, p1, w1t, bng, bnb, w2t, bg, bb, pwt, h_out, part):
  ho = _mlp_bn(p0[...] + p1[...], w1t[...], bng[...], bnb[...],
               w2t[...], bg[...], bb[...])
  h_out[...] = ho
  pooled = jnp.sum(ho, axis=0, keepdims=True)
  part[...] = jnp.dot(pooled, pwt[...], preferred_element_type=jnp.float32)


def _tc_layer0_body(x, p0, p1, w1t, bng, bnb, w2t, bg, bb, pw0t, pwt,
                    h_out, part0, part1):
  px = jnp.sum(x[...], axis=0, keepdims=True)
  part0[...] = jnp.dot(px, pw0t[...], preferred_element_type=jnp.float32)
  ho = _mlp_bn(p0[...] + p1[...], w1t[...], bng[...], bnb[...],
               w2t[...], bg[...], bb[...])
  h_out[...] = ho
  pooled = jnp.sum(ho, axis=0, keepdims=True)
  part1[...] = jnp.dot(pooled, pwt[...], preferred_element_type=jnp.float32)


def _tc_layer(p0, p1, w1t, bng, bnb, w2t, bg, bb, pwt):
  out_d = w2t.shape[1]
  return pl.pallas_call(
      _tc_layer_body,
      out_shape=(jax.ShapeDtypeStruct((_N, out_d), jnp.float32),
                 jax.ShapeDtypeStruct((1, _OUT), jnp.float32)),
  )(p0, p1, w1t, bng, bnb, w2t, bg, bb, pwt)


def _tc_layer0(x, p0, p1, w1t, bng, bnb, w2t, bg, bb, pw0t, pwt):
  return pl.pallas_call(
      _tc_layer0_body,
      out_shape=(jax.ShapeDtypeStruct((_N, _H), jnp.float32),
                 jax.ShapeDtypeStruct((1, _OUT), jnp.float32),
                 jax.ShapeDtypeStruct((1, _OUT), jnp.float32)),
  )(x, p0, p1, w1t, bng, bnb, w2t, bg, bb, pw0t, pwt)


def _final_body(parts, biases, out):
  score = jnp.sum(parts[...] + biases[...], axis=0, keepdims=True)
  mx = jnp.max(score, axis=-1, keepdims=True)
  sh = score - mx
  out[...] = sh - jnp.log(jnp.sum(jnp.exp(sh), axis=-1, keepdims=True))


def _final(parts, biases):
  return pl.pallas_call(
      _final_body,
      out_shape=jax.ShapeDtypeStruct((1, _OUT), jnp.float32),
  )(parts, biases)


def kernel(x, edge_index, params):
  x = x.astype(jnp.float32)
  src = edge_index[0].astype(jnp.int32).reshape(_NW, _EPT)
  dst = edge_index[1].astype(jnp.int32).reshape(_NW, _EPT)
  src = jnp.pad(src, ((0, 0), (0, _PAD))).reshape(_NW, _NCHUNK, _K)
  dst = jnp.pad(dst, ((0, 0), (0, _PAD)),
                constant_values=_N).reshape(_NW, _NCHUNK, _K)
  zeros = jnp.zeros((_N + _NPAD, _D), jnp.float32)

  parts = []
  h = x
  for i in range(_NUM_GIN):
    out_d = 1 if i == _NUM_GIN - 1 else _H
    p0, p1 = _sc_agg(h, zeros, src, dst)
    w1t = params[f"gin{i}_W1"].T
    w2t = params[f"gin{i}_W2"].T
    bng = params[f"gin{i}_bng"].reshape(1, _H)
    bnb = params[f"gin{i}_bnb"].reshape(1, _H)
    bg = params[f"bn{i}_g"].reshape(1, out_d)
    bb = params[f"bn{i}_b"].reshape(1, out_d)
    pwt = params[f"pred{i + 1}_W"].T
    if i == 0:
      h, part0, part1 = _tc_layer0(x, p0, p1, w1t, bng, bnb, w2t, bg, bb,
                                   params["pred0_W"].T, pwt)
      parts += [part0, part1]
    else:
      h, part = _tc_layer(p0, p1, w1t, bng, bnb, w2t, bg, bb, pwt)
      parts.append(part)

  parts_all = jnp.concatenate(parts, axis=0)
  biases = jnp.stack([params[f"pred{i}_b"] for i in range(_NUM_GIN + 1)],
                     axis=0)
  return _final(parts_all, biases)
